# baseline (speedup 1.0000x reference)
import jax
import jax.numpy as jnp
from jax import lax
from jax.experimental import pallas as pl
from jax.experimental.pallas import tpu as pltpu
from jax.experimental.pallas import tpu_sc as plsc

_K = 64
_G = 8
_V, _H, _W = 4, 96, 160
_NC, _NS, _L = 2, 16, 16
_GPC = _G // _NC
_TPG = _NS // _GPC
_WT = _W // _TPG
_NK = _H // _L
_THR2 = 4.0
_SLOT = 1152


def _zero(ref, nwords):
    z = jnp.zeros((_L,), jnp.float32)
    un = 8 if nwords % (8 * _L) == 0 else 4

    def b(i, _):
        for u in range(un):
            ref[pl.ds((i * un + u) * _L, _L)] = z
        return 0

    lax.fori_loop(0, nwords // (un * _L), b, 0)


def _lane_reduce(acc, red, nw):
    def b(j, _):
        t0 = acc[pl.ds(0 * nw + j * _L, _L)]
        t1 = acc[pl.ds(1 * nw + j * _L, _L)]
        t2 = acc[pl.ds(2 * nw + j * _L, _L)]
        t3 = acc[pl.ds(3 * nw + j * _L, _L)]
        for l in range(4, _L, 4):
            t0 = t0 + acc[pl.ds(l * nw + j * _L, _L)]
            t1 = t1 + acc[pl.ds((l + 1) * nw + j * _L, _L)]
            t2 = t2 + acc[pl.ds((l + 2) * nw + j * _L, _L)]
            t3 = t3 + acc[pl.ds((l + 3) * nw + j * _L, _L)]
        red[pl.ds(j * _L, _L)] = (t0 + t1) + (t2 + t3)
        return 0

    lax.fori_loop(0, nw // _L, b, 0)


def _rsqrt(x):
    bits = lax.bitcast_convert_type(x, jnp.int32)
    y = lax.bitcast_convert_type(jnp.int32(0x5F3759DF) - (bits >> 1),
                                 jnp.float32)
    for _ in range(4):
        y = y * (1.5 - 0.5 * x * y * y)
    return y


def _body(c_t, off_t, op_t, sc_t, rot_t, fd_t, cf_t, af_t, mo_t, ids_t,
          oc, ooff, oop, osc, orot, ofd, ocf, oaf, omo,
          ids_l, sel_l, cbufA, cbufB, rbufA, rbufB, abufA, abufB, idsbuf,
          acc, red, tmp, mean1_v, recip_v, meanf_v, shared,
          sin0, sin1, sout0, sout1):
    core = lax.axis_index("c")
    s = lax.axis_index("s")
    g = core * _GPC + s // _TPG
    w0 = (s % _TPG) * _WT
    qbase = (s // _TPG) * _TPG
    iota = lax.iota(jnp.int32, _L)
    ones = jnp.ones((_L,), jnp.float32)
    zerov = jnp.zeros((_L,), jnp.int32)

    def combine(nw):
        for t in range(_TPG):
            pltpu.sync_copy(shared.at[pl.ds((qbase + t) * _SLOT, nw)],
                            tmp.at[pl.ds(t * nw, nw)])

        def b(j, _):
            tot = tmp[pl.ds(j * _L, _L)]
            for t in range(1, _TPG):
                tot = tot + tmp[pl.ds(t * nw + j * _L, _L)]
            red[pl.ds(j * _L, _L)] = tot
            return 0

        lax.fori_loop(0, nw // _L, b, 0)

    def share_combine(nw, first=False):
        if not first:
            plsc.subcore_barrier()
        pltpu.sync_copy(red.at[pl.ds(0, nw)], shared.at[pl.ds(s * _SLOT, nw)])
        plsc.subcore_barrier()
        combine(nw)

    _zero(acc, _L * 320)

    def ph0(v, _):
        def kl(k, _):
            pltpu.sync_copy(ids_t.at[g, v, pl.ds(k * _L, _L), :],
                            idsbuf)

            def dwl(dw, _):
                col = (v * _WT + dw) * _H
                vec = plsc.load_gather(idsbuf, [iota, zerov + (w0 + dw)])
                ids_l[pl.ds(col + k * _L, _L)] = vec
                plsc.addupdate_scatter(acc, [iota * 320 + vec * 5 + 3], ones)
                return 0

            lax.fori_loop(0, _WT, dwl, 0)
            return 0

        lax.fori_loop(0, _NK, kl, 0)
        return 0

    lax.fori_loop(0, _V, ph0, 0)

    def stream_acc(fref, bufs, nchunks, slc, chunk_body):
        bufA, bufB = bufs
        pltpu.async_copy(slc(fref, 0), bufA, sin0)
        pltpu.async_copy(slc(fref, 1), bufB, sin1)

        def aj(j, _):
            for u, (buf, sem) in enumerate(((bufA, sin0), (bufB, sin1))):
                t = 2 * j + u
                pltpu.make_async_copy(slc(fref, t), buf, sem).wait()
                chunk_body(buf, t)

                @pl.when(t + 2 < nchunks)
                def _():
                    pltpu.async_copy(slc(fref, t + 2), buf, sem)
            return 0

        lax.fori_loop(0, nchunks // 2, aj, 0)

    def stream_blend(fref, oref, bufs, nchunks, slc, chunk_body):
        bufA, bufB = bufs
        pltpu.async_copy(slc(fref, 0), bufA, sin0)
        pltpu.async_copy(slc(fref, 1), bufB, sin1)

        def bj(j, _):
            for u, (buf, sem, so) in enumerate(((bufA, sin0, sout0),
                                                (bufB, sin1, sout1))):
                t = 2 * j + u
                pltpu.make_async_copy(slc(fref, t), buf, sem).wait()
                chunk_body(buf, t)
                pltpu.async_copy(buf, slc(oref, t), so)

                @pl.when(t + 2 < nchunks)
                def _():
                    pltpu.make_async_copy(buf, slc(oref, t), so).wait()
                    pltpu.async_copy(slc(fref, t + 2), buf, sem)
            return 0

        lax.fori_loop(0, nchunks // 2, bj, 0)
        pltpu.make_async_copy(bufA, slc(oref, nchunks - 2), sout0).wait()
        pltpu.make_async_copy(bufB, slc(oref, nchunks - 1), sout1).wait()

    def f3_slc(ref, t):
        v = (t * 86) >> 8
        chv = t - v * 3
        return ref.at[g, v, chv, pl.ds(w0, _WT), :]

    def f3_decode(t):
        v = (t * 86) >> 8
        return v, t - v * 3

    def p1_body(buf, t):
        v, chv = f3_decode(t)

        def dwl(dw, _):
            col = (v * _WT + dw) * _H
            for k in range(_NK):
                idv = ids_l[pl.ds(col + k * _L, _L)]
                val = buf[dw, pl.ds(k * _L, _L)]
                plsc.addupdate_scatter(acc, [iota * 320 + idv * 5 + chv],
                                       val)
            return 0

        lax.fori_loop(0, _WT, dwl, 0)

    stream_acc(c_t, (cbufA, cbufB), 12, f3_slc, p1_body)
    _lane_reduce(acc, red, 320)
    share_combine(320, first=True)
    for tc in range(4):
        t5 = (iota + tc * _L) * 5
        cnt = plsc.load_gather(red, [t5 + 3])
        d = jnp.maximum(cnt, 1.0)
        for ch in range(3):
            sm = plsc.load_gather(red, [t5 + ch])
            plsc.store_scatter(mean1_v, [t5 + ch], sm / d)
        plsc.store_scatter(mean1_v, [t5 + 3],
                           jnp.where(cnt >= 2.0, 1.0, 0.0))

    _zero(acc, _L * _K)

    def p2a_body(buf, t):
        v, chv = f3_decode(t)
        is_last = chv == 2

        def dwl(dw, _):
            col = (v * _WT + dw) * _H
            for k in range(_NK):
                idv = ids_l[pl.ds(col + k * _L, _L)]
                t5 = idv * 5
                cv = buf[dw, pl.ds(k * _L, _L)]
                mv = plsc.load_gather(mean1_v, [t5 + chv])
                dd = cv - mv
                prev = sel_l[pl.ds(col + k * _L, _L)]
                d2 = jnp.where(chv == 0, 0.0, prev) + dd * dd
                flag = plsc.load_gather(mean1_v, [t5 + 3])
                selv = jnp.where((d2 <= _THR2) & (flag > 0.5), 1.0, 0.0)
                outv = jnp.where(is_last, selv, d2)
                sel_l[pl.ds(col + k * _L, _L)] = outv
                plsc.addupdate_scatter(acc, [iota * _K + idv],
                                       jnp.where(is_last, selv, 0.0))
            return 0

        lax.fori_loop(0, _WT, dwl, 0)

    stream_acc(c_t, (cbufA, cbufB), 12, f3_slc, p2a_body)
    _lane_reduce(acc, red, _K)
    share_combine(_K)

    def mkrec(j, _):
        recip_v[pl.ds(j * _L, _L)] = 1.0 / jnp.maximum(
            red[pl.ds(j * _L, _L)], 1.0)
        return 0

    lax.fori_loop(0, _K // _L, mkrec, 0)

    def mkmean(SP, nch):
        for tc in range(4):
            trk = (iota + tc * _L) * SP
            rc = plsc.load_gather(recip_v, [iota + tc * _L])
            for ch in range(nch):
                sm = plsc.load_gather(red, [trk + ch])
                plsc.store_scatter(meanf_v, [trk + ch], sm * rc)

    def field_f3(fref, oref):
        nw = _K * 5
        _zero(acc, _L * nw)

        def acc_body(buf, t):
            v, chv = f3_decode(t)

            def dwl(dw, _):
                col = (v * _WT + dw) * _H
                for k in range(_NK):
                    idv = ids_l[pl.ds(col + k * _L, _L)]
                    sv = sel_l[pl.ds(col + k * _L, _L)]
                    val = buf[dw, pl.ds(k * _L, _L)]
                    plsc.addupdate_scatter(acc, [iota * nw + idv * 5 + chv],
                                           val * sv)
                return 0

            lax.fori_loop(0, _WT, dwl, 0)

        stream_acc(fref, (cbufA, cbufB), 12, f3_slc, acc_body)
        _lane_reduce(acc, red, nw)
        share_combine(nw)
        mkmean(5, 3)

        def blend_body(buf, t):
            v, chv = f3_decode(t)

            def dwl(dw, _):
                col = (v * _WT + dw) * _H
                for k in range(_NK):
                    idv = ids_l[pl.ds(col + k * _L, _L)]
                    selb = sel_l[pl.ds(col + k * _L, _L)] > 0.5
                    mv = plsc.load_gather(meanf_v, [idv * 5 + chv])
                    val = buf[dw, pl.ds(k * _L, _L)]
                    buf[dw, pl.ds(k * _L, _L)] = jnp.where(selb, mv, val)
                return 0

            lax.fori_loop(0, _WT, dwl, 0)

        stream_blend(fref, oref, (cbufA, cbufB), 12, f3_slc, blend_body)

    def field_op1(fref, oref):
        _zero(acc, _L * _K)

        def slc(ref, t):
            return ref.at[g, t, pl.ds(w0, _WT), :]

        def acc_body(buf, t):
            def dwl(dw, _):
                col = (t * _WT + dw) * _H
                for k in range(_NK):
                    idv = ids_l[pl.ds(col + k * _L, _L)]
                    sv = sel_l[pl.ds(col + k * _L, _L)]
                    val = buf[dw, pl.ds(k * _L, _L)]
                    plsc.addupdate_scatter(acc, [iota * _K + idv], val * sv)
                return 0

            lax.fori_loop(0, _WT, dwl, 0)

        stream_acc(fref, (cbufA, cbufB), 4, slc, acc_body)
        _lane_reduce(acc, red, _K)
        share_combine(_K)
        mkmean(1, 1)

        def blend_body(buf, t):
            def dwl(dw, _):
                col = (t * _WT + dw) * _H
                for k in range(_NK):
                    idv = ids_l[pl.ds(col + k * _L, _L)]
                    selb = sel_l[pl.ds(col + k * _L, _L)] > 0.5
                    mv = plsc.load_gather(meanf_v, [idv])
                    val = buf[dw, pl.ds(k * _L, _L)]
                    buf[dw, pl.ds(k * _L, _L)] = jnp.where(selb, mv, val)
                return 0

            lax.fori_loop(0, _WT, dwl, 0)

        stream_blend(fref, oref, (cbufA, cbufB), 4, slc, blend_body)

    def field_cmin(fref, oref, bufs, nch, do_norm):
        SP = {4: 5, 16: 17}[nch]
        nw = _K * SP
        wcw = _WT // 4
        _zero(acc, _L * nw)

        def slc(ref, t):
            return ref.at[g, t >> 2, pl.ds(w0 + (t & 3) * wcw, wcw)]

        def acc_body(buf, t):
            def dwl(dw, _):
                col = ((t >> 2) * _WT + (t & 3) * wcw + dw) * _H

                def kbody(k):
                    idv = ids_l[pl.ds(col + k * _L, _L)]
                    sv = sel_l[pl.ds(col + k * _L, _L)]
                    ab = iota * nw + idv * SP
                    for ch in range(nch):
                        val = buf[dw, ch, pl.ds(k * _L, _L)]
                        plsc.addupdate_scatter(acc, [ab + ch], val * sv)

                if nch <= 4:
                    for k in range(_NK):
                        kbody(k)
                else:
                    def kl(k, _):
                        kbody(k)
                        return 0

                    lax.fori_loop(0, _NK, kl, 0)
                return 0

            lax.fori_loop(0, wcw, dwl, 0)

        stream_acc(fref, bufs, 16, slc, acc_body)
        _lane_reduce(acc, red, nw)
        share_combine(nw)
        mkmean(SP, nch)
        if do_norm:
            for tc in range(4):
                t5 = (iota + tc * _L) * 5
                sq = jnp.zeros((_L,), jnp.float32)
                for ch in range(4):
                    mv = plsc.load_gather(meanf_v, [t5 + ch])
                    sq = sq + mv * mv
                r = _rsqrt(jnp.maximum(sq, 1e-24))
                for ch in range(4):
                    mv = plsc.load_gather(meanf_v, [t5 + ch])
                    plsc.store_scatter(meanf_v, [t5 + ch], mv * r)

        def blend_body(buf, t):
            def dwl(dw, _):
                col = ((t >> 2) * _WT + (t & 3) * wcw + dw) * _H

                def kbody(k):
                    idv = ids_l[pl.ds(col + k * _L, _L)]
                    selb = sel_l[pl.ds(col + k * _L, _L)] > 0.5
                    mb = idv * SP
                    for ch in range(nch):
                        mv = plsc.load_gather(meanf_v, [mb + ch])
                        val = buf[dw, ch, pl.ds(k * _L, _L)]
                        buf[dw, ch, pl.ds(k * _L, _L)] = jnp.where(
                            selb, mv, val)

                if nch <= 4:
                    for k in range(_NK):
                        kbody(k)
                else:
                    def kl(k, _):
                        kbody(k)
                        return 0

                    lax.fori_loop(0, _NK, kl, 0)
                return 0

            lax.fori_loop(0, wcw, dwl, 0)

        stream_blend(fref, oref, bufs, 16, slc, blend_body)

    field_f3(c_t, oc)
    field_f3(off_t, ooff)
    field_op1(op_t, oop)
    field_f3(sc_t, osc)
    field_cmin(rot_t, orot, (rbufA, rbufB), 4, True)
    field_f3(fd_t, ofd)
    field_op1(cf_t, ocf)
    field_cmin(af_t, oaf, (abufA, abufB), 16, False)
    field_cmin(mo_t, omo, (abufA, abufB), 16, False)


def _make_kernel():
    f32 = jnp.float32
    return pl.kernel(
        _body,
        out_type=(
            jax.ShapeDtypeStruct((_G, _V, 3, _W, _H), f32),
            jax.ShapeDtypeStruct((_G, _V, 3, _W, _H), f32),
            jax.ShapeDtypeStruct((_G, _V, _W, _H), f32),
            jax.ShapeDtypeStruct((_G, _V, 3, _W, _H), f32),
            jax.ShapeDtypeStruct((_G, _V, _W, 4, _H), f32),
            jax.ShapeDtypeStruct((_G, _V, 3, _W, _H), f32),
            jax.ShapeDtypeStruct((_G, _V, _W, _H), f32),
            jax.ShapeDtypeStruct((_G, _V, _W, 16, _H), f32),
            jax.ShapeDtypeStruct((_G, _V, _W, 16, _H), f32),
        ),
        mesh=plsc.VectorSubcoreMesh(core_axis_name="c", subcore_axis_name="s",
                                    num_cores=_NC, num_subcores=_NS),
        compiler_params=pltpu.CompilerParams(needs_layout_passes=False),
        scratch_types=[
            pltpu.VMEM((_V * _WT * _H,), jnp.int32),
            pltpu.VMEM((_V * _WT * _H,), f32),
            pltpu.VMEM((_WT, _H), f32),
            pltpu.VMEM((_WT, _H), f32),
            pltpu.VMEM((_WT // 4, 4, _H), f32),
            pltpu.VMEM((_WT // 4, 4, _H), f32),
            pltpu.VMEM((_WT // 4, 16, _H), f32),
            pltpu.VMEM((_WT // 4, 16, _H), f32),
            pltpu.VMEM((_L, _W), jnp.int32),
            pltpu.VMEM((_L * 1088,), f32),
            pltpu.VMEM((1088,), f32),
            pltpu.VMEM((4352,), f32),
            pltpu.VMEM((320,), f32),
            pltpu.VMEM((_K,), f32),
            pltpu.VMEM((1088,), f32),
            pltpu.VMEM_SHARED((_NS * _SLOT,), f32),
            pltpu.SemaphoreType.DMA,
            pltpu.SemaphoreType.DMA,
            pltpu.SemaphoreType.DMA,
            pltpu.SemaphoreType.DMA,
        ],
    )


def kernel(center, offset, opacity, scale, rotation, feat_dc, confidence,
           instance_affinity, motion_code, global_track_id):
    def f3(x):
        return x.reshape(_G, _V, _H, _W, 3).transpose(0, 1, 4, 3, 2)

    def cmin(x, c):
        return x.reshape(_G, _V, _H, _W, c).transpose(0, 1, 3, 4, 2)

    def op1(x):
        return x.reshape(_G, _V, _H, _W).transpose(0, 1, 3, 2)

    ins = (f3(center), f3(offset), op1(opacity), f3(scale),
           cmin(rotation, 4), f3(feat_dc), op1(confidence),
           cmin(instance_affinity, 16), cmin(motion_code, 16),
           global_track_id.reshape(_G, _V, _H, _W))
    o = _make_kernel()(*ins)

    def inv_f3(x, ref):
        return x.transpose(0, 1, 4, 3, 2).reshape(ref.shape)

    def inv_cmin(x, ref):
        return x.transpose(0, 1, 4, 2, 3).reshape(ref.shape)

    def inv_op1(x, ref):
        return x.transpose(0, 1, 3, 2).reshape(ref.shape)

    return (inv_f3(o[0], center), inv_f3(o[1], offset), inv_op1(o[2], opacity),
            inv_f3(o[3], scale), inv_cmin(o[4], rotation),
            inv_f3(o[5], feat_dc), inv_op1(o[6], confidence),
            inv_cmin(o[7], instance_affinity), inv_cmin(o[8], motion_code))


# speedup vs baseline: 1.1018x; 1.1018x over previous
import jax
import jax.numpy as jnp
from jax import lax
from jax.experimental import pallas as pl
from jax.experimental.pallas import tpu as pltpu
from jax.experimental.pallas import tpu_sc as plsc

_K = 64
_G = 8
_V, _H, _W = 4, 96, 160
_NC, _NS, _L = 2, 16, 16
_GPC = _G // _NC
_TPG = _NS // _GPC
_WT = _W // _TPG
_NK = _H // _L
_THR2 = 4.0


def _zero(ref, nwords):
    z = jnp.zeros((_L,), jnp.float32)
    un = 8 if nwords % (8 * _L) == 0 else 4

    def b(i, _):
        for u in range(un):
            ref[pl.ds((i * un + u) * _L, _L)] = z
        return 0

    lax.fori_loop(0, nwords // (un * _L), b, 0)


def _lane_reduce(acc, red, nw):
    def b(j, _):
        t0 = acc[pl.ds(0 * nw + j * _L, _L)]
        t1 = acc[pl.ds(1 * nw + j * _L, _L)]
        t2 = acc[pl.ds(2 * nw + j * _L, _L)]
        t3 = acc[pl.ds(3 * nw + j * _L, _L)]
        for l in range(4, _L, 4):
            t0 = t0 + acc[pl.ds(l * nw + j * _L, _L)]
            t1 = t1 + acc[pl.ds((l + 1) * nw + j * _L, _L)]
            t2 = t2 + acc[pl.ds((l + 2) * nw + j * _L, _L)]
            t3 = t3 + acc[pl.ds((l + 3) * nw + j * _L, _L)]
        red[pl.ds(j * _L, _L)] = (t0 + t1) + (t2 + t3)
        return 0

    lax.fori_loop(0, nw // _L, b, 0)


def _combine(shared, tmp, red, qbase, nw):
    for t in range(_TPG):
        pltpu.sync_copy(shared.at[pl.ds((qbase + t) * 1152, nw)],
                        tmp.at[pl.ds(t * nw, nw)])

    def b(j, _):
        tot = tmp[pl.ds(j * _L, _L)]
        for t in range(1, _TPG):
            tot = tot + tmp[pl.ds(t * nw + j * _L, _L)]
        red[pl.ds(j * _L, _L)] = tot
        return 0

    lax.fori_loop(0, nw // _L, b, 0)


def _rsqrt(x):
    bits = lax.bitcast_convert_type(x, jnp.int32)
    y = lax.bitcast_convert_type(jnp.int32(0x5F3759DF) - (bits >> 1),
                                 jnp.float32)
    for _ in range(4):
        y = y * (1.5 - 0.5 * x * y * y)
    return y


def _share_combine(shared, tmp, red, s, qbase, nw):
    plsc.subcore_barrier()
    pltpu.sync_copy(red.at[pl.ds(0, nw)], shared.at[pl.ds(s * 1152, nw)])
    plsc.subcore_barrier()
    _combine(shared, tmp, red, qbase, nw)


def _body(c_t, off_t, op_t, sc_t, rot_t, fd_t, cf_t, af_t, mo_t, ids_t,
          oc, ooff, oop, osc, orot, ofd, ocf, oaf, omo,
          ids_l, sel_l, cbuf, rbuf, abuf, abuf2, idsbuf,
          acc, red, tmp, mean1_v, recip_v, meanf_v, shared,
          sin0, sin1, sout0, sout1):
    core = lax.axis_index("c")
    s = lax.axis_index("s")
    g = core * _GPC + s // _TPG
    w0 = (s % _TPG) * _WT
    qbase = (s // _TPG) * _TPG
    iota = lax.iota(jnp.int32, _L)
    ones = jnp.ones((_L,), jnp.float32)
    zerov = jnp.zeros((_L,), jnp.int32)

    def ph0(v, _):
        def kl(k, _):
            pltpu.sync_copy(ids_t.at[g, v, pl.ds(k * _L, _L), :],
                            idsbuf)

            def dwl(dw, _):
                col = (v * _WT + dw) * _H
                vec = plsc.load_gather(idsbuf, [iota, zerov + (w0 + dw)])
                ids_l[pl.ds(col + k * _L, _L)] = vec
                return 0

            lax.fori_loop(0, _WT, dwl, 0)
            return 0

        lax.fori_loop(0, _NK, kl, 0)
        return 0

    lax.fori_loop(0, _V, ph0, 0)

    _zero(acc, _L * 320)

    def p1(v, _):
        pltpu.sync_copy(c_t.at[g, v, :, pl.ds(w0, _WT), :], cbuf)

        def dwl(dw, _):
            col = (v * _WT + dw) * _H
            for k in range(_NK):
                idv = ids_l[pl.ds(col + k * _L, _L)]
                ab = iota * 320 + idv * 5
                for ch in range(3):
                    val = cbuf[ch, dw, pl.ds(k * _L, _L)]
                    plsc.addupdate_scatter(acc, [ab + ch], val)
                plsc.addupdate_scatter(acc, [ab + 3], ones)
            return 0

        lax.fori_loop(0, _WT, dwl, 0)
        return 0

    lax.fori_loop(0, _V, p1, 0)
    _lane_reduce(acc, red, 320)
    pltpu.sync_copy(red.at[pl.ds(0, 320)], shared.at[pl.ds(s * 1152, 320)])
    plsc.subcore_barrier()
    _combine(shared, tmp, red, qbase, 320)
    for tc in range(4):
        t4 = (iota + tc * _L) * 5
        cnt = plsc.load_gather(red, [t4 + 3])
        d = jnp.maximum(cnt, 1.0)
        for ch in range(3):
            sm = plsc.load_gather(red, [t4 + ch])
            plsc.store_scatter(mean1_v, [t4 + ch], sm / d)
        plsc.store_scatter(mean1_v, [t4 + 3],
                           jnp.where(cnt >= 2.0, 1.0, 0.0))

    _zero(acc, _L * _K)

    def p2a(v, _):
        pltpu.sync_copy(c_t.at[g, v, :, pl.ds(w0, _WT), :], cbuf)

        def dwl(dw, _):
            col = (v * _WT + dw) * _H
            for k in range(_NK):
                idv = ids_l[pl.ds(col + k * _L, _L)]
                t4 = idv * 5
                flag = plsc.load_gather(mean1_v, [t4 + 3])
                d2 = jnp.zeros((_L,), jnp.float32)
                for ch in range(3):
                    cv = cbuf[ch, dw, pl.ds(k * _L, _L)]
                    mv = plsc.load_gather(mean1_v, [t4 + ch])
                    dd = cv - mv
                    d2 = d2 + dd * dd
                selv = jnp.where((d2 <= _THR2) & (flag > 0.5), 1.0, 0.0)
                sel_l[pl.ds(col + k * _L, _L)] = selv
                plsc.addupdate_scatter(acc, [iota * _K + idv], selv)
            return 0

        lax.fori_loop(0, _WT, dwl, 0)
        return 0

    lax.fori_loop(0, _V, p2a, 0)
    _lane_reduce(acc, red, _K)
    _share_combine(shared, tmp, red, s, qbase, _K)

    def mkrec(j, _):
        recip_v[pl.ds(j * _L, _L)] = 1.0 / jnp.maximum(
            red[pl.ds(j * _L, _L)], 1.0)
        return 0

    lax.fori_loop(0, _K // _L, mkrec, 0)

    def mkmean(nw, SP, nch):
        for tc in range(4):
            trk = (iota + tc * _L) * SP
            rc = plsc.load_gather(recip_v, [iota + tc * _L])
            for ch in range(nch):
                sm = plsc.load_gather(red, [trk + ch])
                plsc.store_scatter(meanf_v, [trk + ch], sm * rc)

    def field_pass(fref, oref, buf, nch, P, wchunks, do_norm):
        SP = {1: 1, 4: 5, 16: 17}[P]
        nw = _K * SP
        wcw = _WT // wchunks
        _zero(acc, _L * nw)

        def av(v, _):
            def wc_loop(wc, _):
                wbase = w0 + wc * wcw
                if nch == 3:
                    pltpu.sync_copy(fref.at[g, v, :, pl.ds(wbase, wcw), :],
                                    buf)
                elif nch == 1:
                    pltpu.sync_copy(fref.at[g, v, pl.ds(wbase, wcw), :], buf)
                else:
                    pltpu.sync_copy(fref.at[g, v, pl.ds(wbase, wcw)], buf)

                def dwl(dw, _):
                    col = (v * _WT + wc * wcw + dw) * _H

                    def kbody(k):
                        idv = ids_l[pl.ds(col + k * _L, _L)]
                        selv = sel_l[pl.ds(col + k * _L, _L)]
                        ab = iota * nw + idv * SP
                        for ch in range(nch):
                            if nch == 3:
                                val = buf[ch, dw, pl.ds(k * _L, _L)]
                            elif nch == 1:
                                val = buf[dw, pl.ds(k * _L, _L)]
                            else:
                                val = buf[dw, ch, pl.ds(k * _L, _L)]
                            plsc.addupdate_scatter(acc, [ab + ch], val * selv)

                    if nch <= 4:
                        for k in range(_NK):
                            kbody(k)
                    else:
                        def kl(k, _):
                            kbody(k)
                            return 0

                        lax.fori_loop(0, _NK, kl, 0)
                    return 0

                lax.fori_loop(0, wcw, dwl, 0)
                return 0

            lax.fori_loop(0, wchunks, wc_loop, 0)
            return 0

        lax.fori_loop(0, _V, av, 0)
        _lane_reduce(acc, red, nw)
        _share_combine(shared, tmp, red, s, qbase, nw)
        mkmean(nw, SP, nch)
        if do_norm:
            for tc in range(4):
                t4 = (iota + tc * _L) * 5
                sq = jnp.zeros((_L,), jnp.float32)
                for ch in range(4):
                    mv = plsc.load_gather(meanf_v, [t4 + ch])
                    sq = sq + mv * mv
                r = _rsqrt(jnp.maximum(sq, 1e-24))
                for ch in range(4):
                    mv = plsc.load_gather(meanf_v, [t4 + ch])
                    plsc.store_scatter(meanf_v, [t4 + ch], mv * r)

        def bv(v, _):
            def wc_loop(wc, _):
                wbase = w0 + wc * wcw
                if nch == 3:
                    src = fref.at[g, v, :, pl.ds(wbase, wcw), :]
                    dst = oref.at[g, v, :, pl.ds(wbase, wcw), :]
                elif nch == 1:
                    src = fref.at[g, v, pl.ds(wbase, wcw), :]
                    dst = oref.at[g, v, pl.ds(wbase, wcw), :]
                else:
                    src = fref.at[g, v, pl.ds(wbase, wcw)]
                    dst = oref.at[g, v, pl.ds(wbase, wcw)]
                pltpu.sync_copy(src, buf)

                def dwl(dw, _):
                    col = (v * _WT + wc * wcw + dw) * _H

                    def kbody(k):
                        idv = ids_l[pl.ds(col + k * _L, _L)]
                        selb = sel_l[pl.ds(col + k * _L, _L)] > 0.5
                        mb = idv * SP
                        for ch in range(nch):
                            mv = plsc.load_gather(meanf_v, [mb + ch])
                            if nch == 3:
                                val = buf[ch, dw, pl.ds(k * _L, _L)]
                                buf[ch, dw, pl.ds(k * _L, _L)] = jnp.where(
                                    selb, mv, val)
                            elif nch == 1:
                                val = buf[dw, pl.ds(k * _L, _L)]
                                buf[dw, pl.ds(k * _L, _L)] = jnp.where(
                                    selb, mv, val)
                            else:
                                val = buf[dw, ch, pl.ds(k * _L, _L)]
                                buf[dw, ch, pl.ds(k * _L, _L)] = jnp.where(
                                    selb, mv, val)

                    if nch <= 4:
                        for k in range(_NK):
                            kbody(k)
                    else:
                        def kl(k, _):
                            kbody(k)
                            return 0

                        lax.fori_loop(0, _NK, kl, 0)
                    return 0

                lax.fori_loop(0, wcw, dwl, 0)
                pltpu.sync_copy(buf, dst)
                return 0

            lax.fori_loop(0, wchunks, wc_loop, 0)
            return 0

        lax.fori_loop(0, _V, bv, 0)

    def a16_src(fref, t):
        return fref.at[g, t >> 2, pl.ds(w0 + (t & 3) * 10, 10)]

    def a16_pass(fref, oref):
        nw = _K * 17
        _zero(acc, _L * nw)

        def process(buf, t, blend):
            def dwl(dw, _):
                col = ((t >> 2) * _WT + (t & 3) * 10 + dw) * _H

                def kl(k, _):
                    idv = ids_l[pl.ds(col + k * _L, _L)]
                    sv = sel_l[pl.ds(col + k * _L, _L)]
                    if blend:
                        selb = sv > 0.5
                        mb = idv * 17
                        for ch in range(16):
                            mv = plsc.load_gather(meanf_v, [mb + ch])
                            val = buf[dw, ch, pl.ds(k * _L, _L)]
                            buf[dw, ch, pl.ds(k * _L, _L)] = jnp.where(
                                selb, mv, val)
                    else:
                        ab = iota * nw + idv * 17
                        for ch in range(16):
                            val = buf[dw, ch, pl.ds(k * _L, _L)]
                            plsc.addupdate_scatter(acc, [ab + ch], val * sv)
                    return 0

                lax.fori_loop(0, _NK, kl, 0)
                return 0

            lax.fori_loop(0, 10, dwl, 0)

        pltpu.async_copy(a16_src(fref, 0), abuf, sin0)
        pltpu.async_copy(a16_src(fref, 1), abuf2, sin1)

        def aj(j, _):
            for u, (buf, sem) in enumerate(((abuf, sin0), (abuf2, sin1))):
                t = 2 * j + u
                pltpu.make_async_copy(a16_src(fref, t), buf, sem).wait()
                process(buf, t, False)

                @pl.when(t + 2 < 16)
                def _():
                    pltpu.async_copy(a16_src(fref, t + 2), buf, sem)
            return 0

        lax.fori_loop(0, 8, aj, 0)
        _lane_reduce(acc, red, nw)
        _share_combine(shared, tmp, red, s, qbase, nw)
        mkmean(nw, 17, 16)

        pltpu.async_copy(a16_src(fref, 0), abuf, sin0)
        pltpu.async_copy(a16_src(fref, 1), abuf2, sin1)

        def bj(j, _):
            for u, (buf, sem, so) in enumerate(((abuf, sin0, sout0),
                                                (abuf2, sin1, sout1))):
                t = 2 * j + u
                pltpu.make_async_copy(a16_src(fref, t), buf, sem).wait()
                process(buf, t, True)
                pltpu.async_copy(buf, a16_src(oref, t), so)

                @pl.when(t + 2 < 16)
                def _():
                    pltpu.make_async_copy(buf, a16_src(oref, t), so).wait()
                    pltpu.async_copy(a16_src(fref, t + 2), buf, sem)
            return 0

        lax.fori_loop(0, 8, bj, 0)
        pltpu.make_async_copy(abuf, a16_src(oref, 14), sout0).wait()
        pltpu.make_async_copy(abuf2, a16_src(oref, 15), sout1).wait()

    def op1_pass(fref, oref):
        bufA, bufB = cbuf.at[0], cbuf.at[1]
        _zero(acc, _L * _K)

        def slc(ref, t):
            return ref.at[g, t, pl.ds(w0, _WT), :]

        def acc_body(buf, t):
            def dwl(dw, _):
                col = (t * _WT + dw) * _H
                for k in range(_NK):
                    idv = ids_l[pl.ds(col + k * _L, _L)]
                    sv = sel_l[pl.ds(col + k * _L, _L)]
                    val = buf[dw, pl.ds(k * _L, _L)]
                    plsc.addupdate_scatter(acc, [iota * _K + idv], val * sv)
                return 0

            lax.fori_loop(0, _WT, dwl, 0)

        pltpu.async_copy(slc(fref, 0), bufA, sin0)
        pltpu.async_copy(slc(fref, 1), bufB, sin1)

        def aj(j, _):
            for u, (buf, sem) in enumerate(((bufA, sin0), (bufB, sin1))):
                t = 2 * j + u
                pltpu.make_async_copy(slc(fref, t), buf, sem).wait()
                acc_body(buf, t)

                @pl.when(t + 2 < 4)
                def _():
                    pltpu.async_copy(slc(fref, t + 2), buf, sem)
            return 0

        lax.fori_loop(0, 2, aj, 0)
        _lane_reduce(acc, red, _K)
        _share_combine(shared, tmp, red, s, qbase, _K)
        mkmean(_K, 1, 1)

        def blend_body(buf, t):
            def dwl(dw, _):
                col = (t * _WT + dw) * _H
                for k in range(_NK):
                    idv = ids_l[pl.ds(col + k * _L, _L)]
                    selb = sel_l[pl.ds(col + k * _L, _L)] > 0.5
                    mv = plsc.load_gather(meanf_v, [idv])
                    val = buf[dw, pl.ds(k * _L, _L)]
                    buf[dw, pl.ds(k * _L, _L)] = jnp.where(selb, mv, val)
                return 0

            lax.fori_loop(0, _WT, dwl, 0)

        pltpu.async_copy(slc(fref, 0), bufA, sin0)
        pltpu.async_copy(slc(fref, 1), bufB, sin1)

        def bj(j, _):
            for u, (buf, sem, so) in enumerate(((bufA, sin0, sout0),
                                                (bufB, sin1, sout1))):
                t = 2 * j + u
                pltpu.make_async_copy(slc(fref, t), buf, sem).wait()
                blend_body(buf, t)
                pltpu.async_copy(buf, slc(oref, t), so)

                @pl.when(t + 2 < 4)
                def _():
                    pltpu.make_async_copy(buf, slc(oref, t), so).wait()
                    pltpu.async_copy(slc(fref, t + 2), buf, sem)
            return 0

        lax.fori_loop(0, 2, bj, 0)
        pltpu.make_async_copy(bufA, slc(oref, 2), sout0).wait()
        pltpu.make_async_copy(bufB, slc(oref, 3), sout1).wait()

    def r4_pass(fref, oref):
        bufA, bufB = rbuf.at[pl.ds(0, 10)], rbuf.at[pl.ds(10, 10)]
        nw = _K * 5
        _zero(acc, _L * nw)

        def slc(ref, t):
            return ref.at[g, t >> 2, pl.ds(w0 + (t & 3) * 10, 10)]

        def body(buf, t, blend):
            def dwl(dw, _):
                col = ((t >> 2) * _WT + (t & 3) * 10 + dw) * _H
                for k in range(_NK):
                    idv = ids_l[pl.ds(col + k * _L, _L)]
                    sv = sel_l[pl.ds(col + k * _L, _L)]
                    if blend:
                        selb = sv > 0.5
                        mb = idv * 5
                        for ch in range(4):
                            mv = plsc.load_gather(meanf_v, [mb + ch])
                            val = buf[dw, ch, pl.ds(k * _L, _L)]
                            buf[dw, ch, pl.ds(k * _L, _L)] = jnp.where(
                                selb, mv, val)
                    else:
                        ab = iota * nw + idv * 5
                        for ch in range(4):
                            val = buf[dw, ch, pl.ds(k * _L, _L)]
                            plsc.addupdate_scatter(acc, [ab + ch], val * sv)
                return 0

            lax.fori_loop(0, 10, dwl, 0)

        pltpu.async_copy(slc(fref, 0), bufA, sin0)
        pltpu.async_copy(slc(fref, 1), bufB, sin1)

        def aj(j, _):
            for u, (buf, sem) in enumerate(((bufA, sin0), (bufB, sin1))):
                t = 2 * j + u
                pltpu.make_async_copy(slc(fref, t), buf, sem).wait()
                body(buf, t, False)

                @pl.when(t + 2 < 16)
                def _():
                    pltpu.async_copy(slc(fref, t + 2), buf, sem)
            return 0

        lax.fori_loop(0, 8, aj, 0)
        _lane_reduce(acc, red, nw)
        _share_combine(shared, tmp, red, s, qbase, nw)
        mkmean(nw, 5, 4)
        for tc in range(4):
            t5 = (iota + tc * _L) * 5
            sq = jnp.zeros((_L,), jnp.float32)
            for ch in range(4):
                mv = plsc.load_gather(meanf_v, [t5 + ch])
                sq = sq + mv * mv
            r = _rsqrt(jnp.maximum(sq, 1e-24))
            for ch in range(4):
                mv = plsc.load_gather(meanf_v, [t5 + ch])
                plsc.store_scatter(meanf_v, [t5 + ch], mv * r)

        pltpu.async_copy(slc(fref, 0), bufA, sin0)
        pltpu.async_copy(slc(fref, 1), bufB, sin1)

        def bj(j, _):
            for u, (buf, sem, so) in enumerate(((bufA, sin0, sout0),
                                                (bufB, sin1, sout1))):
                t = 2 * j + u
                pltpu.make_async_copy(slc(fref, t), buf, sem).wait()
                body(buf, t, True)
                pltpu.async_copy(buf, slc(oref, t), so)

                @pl.when(t + 2 < 16)
                def _():
                    pltpu.make_async_copy(buf, slc(oref, t), so).wait()
                    pltpu.async_copy(slc(fref, t + 2), buf, sem)
            return 0

        lax.fori_loop(0, 8, bj, 0)
        pltpu.make_async_copy(bufA, slc(oref, 14), sout0).wait()
        pltpu.make_async_copy(bufB, slc(oref, 15), sout1).wait()

    field_pass(c_t, oc, cbuf, 3, 4, 1, False)
    field_pass(off_t, ooff, cbuf, 3, 4, 1, False)
    op1_pass(op_t, oop)
    field_pass(sc_t, osc, cbuf, 3, 4, 1, False)
    r4_pass(rot_t, orot)
    field_pass(fd_t, ofd, cbuf, 3, 4, 1, False)
    op1_pass(cf_t, ocf)
    a16_pass(af_t, oaf)
    a16_pass(mo_t, omo)


def _make_kernel():
    f32 = jnp.float32
    return pl.kernel(
        _body,
        out_type=(
            jax.ShapeDtypeStruct((_G, _V, 3, _W, _H), f32),
            jax.ShapeDtypeStruct((_G, _V, 3, _W, _H), f32),
            jax.ShapeDtypeStruct((_G, _V, _W, _H), f32),
            jax.ShapeDtypeStruct((_G, _V, 3, _W, _H), f32),
            jax.ShapeDtypeStruct((_G, _V, _W, 4, _H), f32),
            jax.ShapeDtypeStruct((_G, _V, 3, _W, _H), f32),
            jax.ShapeDtypeStruct((_G, _V, _W, _H), f32),
            jax.ShapeDtypeStruct((_G, _V, _W, 16, _H), f32),
            jax.ShapeDtypeStruct((_G, _V, _W, 16, _H), f32),
        ),
        mesh=plsc.VectorSubcoreMesh(core_axis_name="c", subcore_axis_name="s",
                                    num_cores=_NC, num_subcores=_NS),
        compiler_params=pltpu.CompilerParams(needs_layout_passes=False),
        scratch_types=[
            pltpu.VMEM((_V * _WT * _H,), jnp.int32),
            pltpu.VMEM((_V * _WT * _H,), f32),
            pltpu.VMEM((3, _WT, _H), f32),
            pltpu.VMEM((_WT // 2, 4, _H), f32),
            pltpu.VMEM((_WT // 4, 16, _H), f32),
            pltpu.VMEM((_WT // 4, 16, _H), f32),
            pltpu.VMEM((_L, _W), jnp.int32),
            pltpu.VMEM((_L * 1088,), f32),
            pltpu.VMEM((1088,), f32),
            pltpu.VMEM((4352,), f32),
            pltpu.VMEM((320,), f32),
            pltpu.VMEM((_K,), f32),
            pltpu.VMEM((1088,), f32),
            pltpu.VMEM_SHARED((_NS * 1152,), f32),
            pltpu.SemaphoreType.DMA,
            pltpu.SemaphoreType.DMA,
            pltpu.SemaphoreType.DMA,
            pltpu.SemaphoreType.DMA,
        ],
    )


def kernel(center, offset, opacity, scale, rotation, feat_dc, confidence,
           instance_affinity, motion_code, global_track_id):
    def f3(x):
        return x.reshape(_G, _V, _H, _W, 3).transpose(0, 1, 4, 3, 2)

    def cmin(x, c):
        return x.reshape(_G, _V, _H, _W, c).transpose(0, 1, 3, 4, 2)

    def op1(x):
        return x.reshape(_G, _V, _H, _W).transpose(0, 1, 3, 2)

    ins = (f3(center), f3(offset), op1(opacity), f3(scale),
           cmin(rotation, 4), f3(feat_dc), op1(confidence),
           cmin(instance_affinity, 16), cmin(motion_code, 16),
           global_track_id.reshape(_G, _V, _H, _W))
    o = _make_kernel()(*ins)

    def inv_f3(x, ref):
        return x.transpose(0, 1, 4, 3, 2).reshape(ref.shape)

    def inv_cmin(x, ref):
        return x.transpose(0, 1, 4, 2, 3).reshape(ref.shape)

    def inv_op1(x, ref):
        return x.transpose(0, 1, 3, 2).reshape(ref.shape)

    return (inv_f3(o[0], center), inv_f3(o[1], offset), inv_op1(o[2], opacity),
            inv_f3(o[3], scale), inv_cmin(o[4], rotation),
            inv_f3(o[5], feat_dc), inv_op1(o[6], confidence),
            inv_cmin(o[7], instance_affinity), inv_cmin(o[8], motion_code))
